# Initial kernel scaffold; baseline (speedup 1.0000x reference)
import jax
import jax.numpy as jnp
from jax.experimental import pallas as pl


def kernel(center_word, pos_word, neg_word, in_emb, out_emb):
    raise NotImplementedError("write your pallas kernel here")


# speedup vs baseline: 5.0698x; 5.0698x over previous
import functools

import jax
import jax.numpy as jnp
from jax import lax
from jax.experimental import pallas as pl
from jax.experimental.pallas import tpu as pltpu
from jax.experimental.pallas import tpu_sc as plsc

VOCAB = 100000
D = 128
B = 16384
L = 200
BL = B * L

NC = 2
NS = 16
NW = NC * NS
CH = 128
SUP = 16
SUP_ROWS = CH * SUP
PER_W = BL // NW
N_SUP = PER_W // SUP_ROWS
CHUNK_ROWS_PER_W = PER_W // CH


def _sc_gather_body(cw, pw, nw, in_t, out_t, oc, op, on,
                    idx_v, b0, b1, b2, b3, semg, semw0, semw1):
    wid = lax.axis_index("s") * NC + lax.axis_index("c")
    base_crow = wid * CHUNK_ROWS_PER_W
    banks = ((b0, b1, semw0), (b2, b3, semw1))
    for idx_hbm, table, out_hbm in ((cw, in_t, oc), (pw, out_t, op), (nw, out_t, on)):
        def super_body(j, carry, idx_hbm=idx_hbm, table=table, out_hbm=out_hbm):
            crow0 = base_crow + j * SUP
            row0 = crow0 * CH
            pltpu.sync_copy(idx_hbm.at[pl.ds(crow0, SUP), :], idx_v)
            live_wh = {}
            for g in range(SUP // 2):
                bank = g % 2
                bufa, bufb, semw = banks[bank]
                if g >= 2:
                    for h in live_wh[bank]:
                        h.wait()
                else:
                    @pl.when(j > 0)
                    def _drain_prev_super(bufa=bufa, bufb=bufb, semw=semw, out_hbm=out_hbm):
                        pltpu.make_async_copy(bufa, out_hbm.at[pl.ds(0, CH)], semw).wait()
                        pltpu.make_async_copy(bufb, out_hbm.at[pl.ds(0, CH)], semw).wait()
                c0 = g * 2
                gh = [
                    pltpu.async_copy(table.at[idx_v.at[c0]], bufa, semg),
                    pltpu.async_copy(table.at[idx_v.at[c0 + 1]], bufb, semg),
                ]
                for h in gh:
                    h.wait()
                live_wh[bank] = [
                    pltpu.async_copy(bufa, out_hbm.at[pl.ds(row0 + c0 * CH, CH)], semw),
                    pltpu.async_copy(bufb, out_hbm.at[pl.ds(row0 + (c0 + 1) * CH, CH)], semw),
                ]
            return carry
        lax.fori_loop(0, N_SUP, super_body, 0)
        for bufa, bufb, semw in banks:
            pltpu.make_async_copy(bufa, out_hbm.at[pl.ds(0, CH)], semw).wait()
            pltpu.make_async_copy(bufb, out_hbm.at[pl.ds(0, CH)], semw).wait()


_sc_gather = functools.partial(
    pl.kernel,
    mesh=plsc.VectorSubcoreMesh(core_axis_name="c", subcore_axis_name="s"),
    out_type=[jax.ShapeDtypeStruct((BL, D), jnp.float32)] * 3,
    scratch_types=[
        pltpu.VMEM((SUP, CH), jnp.int32),
        pltpu.VMEM((CH, D), jnp.float32),
        pltpu.VMEM((CH, D), jnp.float32),
        pltpu.VMEM((CH, D), jnp.float32),
        pltpu.VMEM((CH, D), jnp.float32),
        pltpu.SemaphoreType.DMA,
        pltpu.SemaphoreType.DMA,
        pltpu.SemaphoreType.DMA,
    ],
)(_sc_gather_body)


G = 8
NG = B // G


def _logsig_sum(x):
    return jnp.sum(jnp.minimum(x, 0.0) - jnp.log1p(jnp.exp(-jnp.abs(x))))


def _tc_loss_body(c_ref, p_ref, n_ref, out_ref):
    g = pl.program_id(0)

    @pl.when(g == 0)
    def _init():
        out_ref[...] = jnp.zeros((1, 1), jnp.float32)

    total = jnp.float32(0.0)
    for b in range(G):
        c = c_ref[b * L:(b + 1) * L, :]
        p = p_ref[b * L:(b + 1) * L, :]
        n = n_ref[b * L:(b + 1) * L, :]
        dn = (((1,), (1,)), ((), ()))
        ps = lax.dot_general(c, p, dn, preferred_element_type=jnp.float32)
        ns = lax.dot_general(c, n, dn, preferred_element_type=jnp.float32)
        total = total + _logsig_sum(ps) + _logsig_sum(-ns)
    out_ref[...] += jnp.full((1, 1), total, jnp.float32)

    @pl.when(g == NG - 1)
    def _finalize():
        out_ref[...] = -out_ref[...] * (1.0 / float(BL))


def _tc_loss(oc, op, on):
    return pl.pallas_call(
        _tc_loss_body,
        grid=(NG,),
        in_specs=[pl.BlockSpec((G * L, D), lambda i: (i, 0))] * 3,
        out_specs=pl.BlockSpec((1, 1), lambda i: (0, 0)),
        out_shape=jax.ShapeDtypeStruct((1, 1), jnp.float32),
    )(oc, op, on)


def kernel(center_word, pos_word, neg_word, in_emb, out_emb):
    cw = center_word.reshape(BL // CH, CH)
    pw = pos_word.reshape(BL // CH, CH)
    nw = neg_word.reshape(BL // CH, CH)
    oc, op, on = _sc_gather(cw, pw, nw, in_emb, out_emb)
    loss = _tc_loss(oc, op, on)
    return loss[0, 0]
